# baseline (speedup 1.0000x reference)
import functools

import jax
import jax.numpy as jnp
from jax import lax
from jax.experimental import pallas as pl
from jax.experimental.pallas import tpu as pltpu
from jax.experimental.pallas import tpu_sc as plsc

N_USERS = 15000
N_ITEMS = 35000
NN = N_USERS + N_ITEMS
E = 800000
D = 64
NL = 4
B = 4096

NC = 2
NS = 16
LANES = 16
NQ = D // LANES
QPC = NQ // NC
DH = D // NC
RPT = NN // NS
BPT = B // NS

SCH = 4000
NSCH = E // SCH
FL = 2048
BCAP = 392 * FL
MAXBLK = BCAP // FL

_mesh = plsc.VectorSubcoreMesh(core_axis_name="c", subcore_axis_name="s")


@functools.partial(
    pl.kernel,
    out_type=jax.ShapeDtypeStruct((NC, B), jnp.float32),
    mesh=_mesh,
    compiler_params=pltpu.CompilerParams(needs_layout_passes=False,
                                         use_tc_tiling_on_sc=False),
    scratch_types=[
        pltpu.HBM((NQ * NN, LANES), jnp.float32),
        pltpu.HBM((NC * NS * BCAP,), jnp.int32),
        pltpu.HBM((NC * NS * BCAP,), jnp.int32),
        pltpu.HBM((NC * NS * BCAP,), jnp.float32),
        pltpu.VMEM((RPT, LANES), jnp.float32),
        pltpu.VMEM((FL, LANES), jnp.float32),
        pltpu.VMEM((SCH,), jnp.int32),
        pltpu.VMEM((SCH,), jnp.int32),
        pltpu.VMEM((SCH,), jnp.float32),
        pltpu.VMEM((FL + LANES,), jnp.int32),
        pltpu.VMEM((FL + LANES,), jnp.int32),
        pltpu.VMEM((FL + LANES,), jnp.float32),
        pltpu.VMEM((BPT, DH), jnp.float32),
        pltpu.VMEM((BPT, DH), jnp.float32),
        pltpu.VMEM((BPT,), jnp.int32),
        pltpu.VMEM((BPT,), jnp.int32),
        pltpu.VMEM((BPT,), jnp.int32),
        pltpu.VMEM((BPT,), jnp.float32),
        pltpu.SemaphoreType.DMA,
        pltpu.SemaphoreType.DMA,
        pltpu.SemaphoreType.DMA,
    ],
)
def _ltocf_sc(users, items, x0, src, dst, w, out,
              xcur, bsrc, bdst, bw, acc, rows, sidx, didx, wbuf,
              stg_s, stg_d, stg_w, bsu, bsi, uidl, iidl, qid, ob,
              sem1, sem2, sem3):
    c = lax.axis_index("c")
    t = lax.axis_index("s")
    iota = lax.iota(jnp.int32, LANES)
    zv = jnp.zeros((LANES,), jnp.float32)
    rbase = (c * NS + t) * BCAP

    pltpu.sync_copy(users.at[pl.ds(t * BPT, BPT)], uidl)
    pltpu.sync_copy(items.at[pl.ds(t * BPT, BPT)], iidl)

    def _prep(i, carry):
        s = pl.ds(i * LANES, LANES)
        iidl[s] = iidl[s] + N_USERS
        return carry
    lax.fori_loop(0, BPT // LANES, _prep, 0)

    def _zb(i, carry):
        bsu[i, pl.ds(0, LANES)] = zv
        bsu[i, pl.ds(LANES, LANES)] = zv
        bsi[i, pl.ds(0, LANES)] = zv
        bsi[i, pl.ds(LANES, LANES)] = zv
        return carry
    lax.fori_loop(0, BPT, _zb, 0)

    def _scan_chunk(js, carry):
        ebase = js * SCH
        d1 = pltpu.async_copy(src.at[pl.ds(ebase, SCH)], sidx, sem1)
        d2 = pltpu.async_copy(dst.at[pl.ds(ebase, SCH)], didx, sem2)
        d3 = pltpu.async_copy(w.at[pl.ds(ebase, SCH)], wbuf, sem3)
        d1.wait()
        d2.wait()
        d3.wait()

        def _vreg(g, cc):
            staged, nblk = cc
            s = pl.ds(g * LANES, LANES)
            sv = sidx[s]
            dv = didx[s]
            wv = wbuf[s]
            m = (dv & (NS - 1)) == t
            mi = m.astype(jnp.int32)
            rank = plsc.cumsum(mi)
            cnt = rank[15]
            pos = rank + (staged - 1)
            plsc.store_scatter(stg_s, [pos], sv, mask=m)
            plsc.store_scatter(stg_d, [pos], dv >> 4, mask=m)
            plsc.store_scatter(stg_w, [pos], wv, mask=m)
            staged = staged + cnt

            full = staged >= FL
            fs, fn = _flush_when(staged, nblk, full)
            return fs, fn
        return lax.fori_loop(0, SCH // LANES, _vreg, carry)

    def _flush_when(staged, nblk, cond):
        @pl.when(cond)
        def _():
            off = rbase + nblk * FL
            pltpu.sync_copy(stg_s.at[pl.ds(0, FL)], bsrc.at[pl.ds(off, FL)])
            pltpu.sync_copy(stg_d.at[pl.ds(0, FL)], bdst.at[pl.ds(off, FL)])
            pltpu.sync_copy(stg_w.at[pl.ds(0, FL)], bw.at[pl.ds(off, FL)])
            stg_s[pl.ds(0, LANES)] = stg_s[pl.ds(FL, LANES)]
            stg_d[pl.ds(0, LANES)] = stg_d[pl.ds(FL, LANES)]
            stg_w[pl.ds(0, LANES)] = stg_w[pl.ds(FL, LANES)]
        return (jnp.where(cond, staged - FL, staged),
                jnp.where(cond, nblk + 1, nblk))

    staged, nblk = lax.fori_loop(0, NSCH, _scan_chunk,
                                 (jnp.int32(0), jnp.int32(0)))

    padpos = staged + iota
    plsc.store_scatter(stg_s, [padpos], iota)
    plsc.store_scatter(stg_d, [padpos], jnp.zeros((LANES,), jnp.int32))
    plsc.store_scatter(stg_w, [padpos], zv)
    staged = staged + ((LANES - (staged & (LANES - 1))) & (LANES - 1))

    def _pad_cond(cc):
        s, _ = cc
        return (s & (FL - 1)) != 0

    def _pad_body(cc):
        s, nb = cc
        pp = s + iota
        plsc.store_scatter(stg_s, [pp], iota)
        plsc.store_scatter(stg_d, [pp], jnp.zeros((LANES,), jnp.int32))
        plsc.store_scatter(stg_w, [pp], zv)
        return s + LANES, nb
    staged, nblk = lax.while_loop(_pad_cond, _pad_body, (staged, nblk))
    staged, nblk = _flush_when(staged, nblk, staged > 0)

    def _fold(dst_ref, q):
        s = pl.ds(q * LANES, LANES)

        def f(i, carry):
            dst_ref[i, s] = dst_ref[i, s] + rows[i, pl.ds(0, LANES)]
            return carry
        lax.fori_loop(0, BPT, f, 0)

    def _permute_ids(idref, qbase):
        def f(i, carry):
            s = pl.ds(i * LANES, LANES)
            v = idref[s]
            qid[s] = (v & (NS - 1)) * RPT + (v >> 4) + qbase
            return carry
        lax.fori_loop(0, BPT // LANES, f, 0)

    for q in range(QPC):
        qbase = (c * QPC + q) * NN
        _permute_ids(uidl, qbase)
        pltpu.sync_copy(x0.at[qid], rows.at[pl.ds(0, BPT)])
        _fold(bsu, q)
        _permute_ids(iidl, qbase)
        pltpu.sync_copy(x0.at[qid], rows.at[pl.ds(0, BPT)])
        _fold(bsi, q)

    for k in range(NL):
        srctab = x0 if k == 0 else xcur
        for q in range(QPC):
            qbase = (c * QPC + q) * NN

            def _zr(i, carry):
                acc[i, pl.ds(0, LANES)] = zv
                return carry
            lax.fori_loop(0, RPT, _zr, 0)

            def _chunk(j, carry):
                off = rbase + j * FL
                d1 = pltpu.async_copy(bsrc.at[pl.ds(off, FL)],
                                      sidx.at[pl.ds(0, FL)], sem1)
                d2 = pltpu.async_copy(bdst.at[pl.ds(off, FL)],
                                      didx.at[pl.ds(0, FL)], sem2)
                d3 = pltpu.async_copy(bw.at[pl.ds(off, FL)],
                                      wbuf.at[pl.ds(0, FL)], sem3)
                d1.wait()
                d2.wait()
                d3.wait()

                def _off(i, cc):
                    s = pl.ds(i * LANES, LANES)
                    v = sidx[s]
                    sidx[s] = (v & (NS - 1)) * RPT + (v >> 4) + qbase
                    return cc
                lax.fori_loop(0, FL // LANES, _off, 0)

                pltpu.sync_copy(srctab.at[sidx.at[pl.ds(0, FL)]], rows)

                def _edges(g, cc):
                    gb = g * LANES
                    s = pl.ds(gb, LANES)
                    dl = didx[s]
                    wv = wbuf[s]
                    ridx = iota + gb
                    for dcol in range(LANES):
                        cidx = jnp.full((LANES,), dcol, jnp.int32)
                        v = plsc.load_gather(rows, [ridx, cidx])
                        plsc.addupdate_scatter(acc, [dl, cidx], v * wv)
                    return cc
                lax.fori_loop(0, FL // LANES, _edges, 0)
                return carry
            lax.fori_loop(0, nblk, _chunk, 0)
            plsc.subcore_barrier()

            pltpu.sync_copy(acc, xcur.at[pl.ds(qbase + t * RPT, RPT)])
            plsc.subcore_barrier()

            _permute_ids(uidl, qbase)
            pltpu.sync_copy(xcur.at[qid], rows.at[pl.ds(0, BPT)])
            _fold(bsu, q)
            _permute_ids(iidl, qbase)
            pltpu.sync_copy(xcur.at[qid], rows.at[pl.ds(0, BPT)])
            _fold(bsi, q)

    def _dot(g, carry):
        gb = g * LANES
        ridx = iota + gb
        accv = jnp.zeros((LANES,), jnp.float32)
        for dcol in range(DH):
            cidx = jnp.full((LANES,), dcol, jnp.int32)
            uv = plsc.load_gather(bsu, [ridx, cidx])
            iv = plsc.load_gather(bsi, [ridx, cidx])
            accv = accv + uv * iv
        ob[pl.ds(gb, LANES)] = accv * (1.0 / ((NL + 1) * (NL + 1)))
        return carry
    lax.fori_loop(0, BPT // LANES, _dot, 0)

    pltpu.sync_copy(ob, out.at[c, pl.ds(t * BPT, BPT)])


def kernel(users, items, user_emb, item_emb, edge_src, edge_dst, edge_w):
    all_emb = jnp.concatenate([user_emb, item_emb], axis=0)
    p = jnp.arange(NN, dtype=jnp.int32)
    inv = (p % RPT) * NS + p // RPT
    em = all_emb[inv]
    xq = em.reshape(NN, NQ, LANES).transpose(1, 0, 2).reshape(NQ * NN, LANES)
    part = _ltocf_sc(users, items, xq, edge_src, edge_dst, edge_w)
    return part[0] + part[1]


# speedup vs baseline: 1.4474x; 1.4474x over previous
import functools

import jax
import jax.numpy as jnp
from jax import lax
from jax.experimental import pallas as pl
from jax.experimental.pallas import tpu as pltpu
from jax.experimental.pallas import tpu_sc as plsc

N_USERS = 15000
N_ITEMS = 35000
NN = N_USERS + N_ITEMS
E = 800000
D = 64
NL = 4
B = 4096

NC = 2
NS = 16
LANES = 16
NQ = D // LANES
QPC = NQ // NC
DH = D // NC
RPT = NN // NS
BPT = B // NS

SCH = 4000
NSCH = E // SCH
FL = 2048
BCAP = 392 * FL
MAXBLK = BCAP // FL

_mesh = plsc.VectorSubcoreMesh(core_axis_name="c", subcore_axis_name="s")


@functools.partial(
    pl.kernel,
    out_type=jax.ShapeDtypeStruct((NC, B), jnp.float32),
    mesh=_mesh,
    compiler_params=pltpu.CompilerParams(needs_layout_passes=False,
                                         use_tc_tiling_on_sc=False),
    scratch_types=[
        pltpu.HBM((NQ * NN, LANES), jnp.float32),
        pltpu.HBM((NC * NS * BCAP,), jnp.int32),
        pltpu.HBM((NC * NS * BCAP,), jnp.int32),
        pltpu.HBM((NC * NS * BCAP,), jnp.float32),
        pltpu.VMEM((RPT, LANES), jnp.float32),
        pltpu.VMEM((FL, LANES), jnp.float32),
        pltpu.VMEM((SCH,), jnp.int32),
        pltpu.VMEM((SCH,), jnp.int32),
        pltpu.VMEM((SCH,), jnp.float32),
        pltpu.VMEM((FL + LANES,), jnp.int32),
        pltpu.VMEM((FL + LANES,), jnp.int32),
        pltpu.VMEM((FL + LANES,), jnp.float32),
        pltpu.VMEM((BPT, DH), jnp.float32),
        pltpu.VMEM((BPT, DH), jnp.float32),
        pltpu.VMEM((BPT,), jnp.int32),
        pltpu.VMEM((BPT,), jnp.int32),
        pltpu.VMEM((BPT,), jnp.int32),
        pltpu.VMEM((BPT,), jnp.float32),
        pltpu.SemaphoreType.DMA,
        pltpu.SemaphoreType.DMA,
        pltpu.SemaphoreType.DMA,
    ],
)
def _ltocf_sc(users, items, x0, src, dst, w, out,
              xcur, bsrc, bdst, bw, acc, rows, sidx, didx, wbuf,
              stg_s, stg_d, stg_w, bsu, bsi, uidl, iidl, qid, ob,
              sem1, sem2, sem3):
    c = lax.axis_index("c")
    t = lax.axis_index("s")
    iota = lax.iota(jnp.int32, LANES)
    zv = jnp.zeros((LANES,), jnp.float32)
    rbase = (c * NS + t) * BCAP

    pltpu.sync_copy(users.at[pl.ds(t * BPT, BPT)], uidl)
    pltpu.sync_copy(items.at[pl.ds(t * BPT, BPT)], iidl)

    def _prep(i, carry):
        s = pl.ds(i * LANES, LANES)
        iidl[s] = iidl[s] + N_USERS
        return carry
    lax.fori_loop(0, BPT // LANES, _prep, 0)

    def _zb(i, carry):
        bsu[i, pl.ds(0, LANES)] = zv
        bsu[i, pl.ds(LANES, LANES)] = zv
        bsi[i, pl.ds(0, LANES)] = zv
        bsi[i, pl.ds(LANES, LANES)] = zv
        return carry
    lax.fori_loop(0, BPT, _zb, 0)

    def _scan_chunk(js, carry):
        ebase = js * SCH
        d1 = pltpu.async_copy(src.at[pl.ds(ebase, SCH)], sidx, sem1)
        d2 = pltpu.async_copy(dst.at[pl.ds(ebase, SCH)], didx, sem2)
        d3 = pltpu.async_copy(w.at[pl.ds(ebase, SCH)], wbuf, sem3)
        d1.wait()
        d2.wait()
        d3.wait()

        def _vreg(g, cc):
            staged, nblk = cc
            s = pl.ds(g * LANES, LANES)
            sv = sidx[s]
            dv = didx[s]
            wv = wbuf[s]
            m = (dv & (NS - 1)) == t
            mi = m.astype(jnp.int32)
            rank = plsc.cumsum(mi)
            cnt = rank[15]
            pos = rank + (staged - 1)
            plsc.store_scatter(stg_s, [pos], sv, mask=m)
            plsc.store_scatter(stg_d, [pos], dv >> 4, mask=m)
            plsc.store_scatter(stg_w, [pos], wv, mask=m)
            staged = staged + cnt

            full = staged >= FL
            fs, fn = _flush_when(staged, nblk, full)
            return fs, fn
        return lax.fori_loop(0, SCH // LANES, _vreg, carry)

    def _flush_when(staged, nblk, cond):
        @pl.when(cond)
        def _():
            off = rbase + nblk * FL
            pltpu.sync_copy(stg_s.at[pl.ds(0, FL)], bsrc.at[pl.ds(off, FL)])
            pltpu.sync_copy(stg_d.at[pl.ds(0, FL)], bdst.at[pl.ds(off, FL)])
            pltpu.sync_copy(stg_w.at[pl.ds(0, FL)], bw.at[pl.ds(off, FL)])
            stg_s[pl.ds(0, LANES)] = stg_s[pl.ds(FL, LANES)]
            stg_d[pl.ds(0, LANES)] = stg_d[pl.ds(FL, LANES)]
            stg_w[pl.ds(0, LANES)] = stg_w[pl.ds(FL, LANES)]
        return (jnp.where(cond, staged - FL, staged),
                jnp.where(cond, nblk + 1, nblk))

    staged, nblk = lax.fori_loop(0, NSCH, _scan_chunk,
                                 (jnp.int32(0), jnp.int32(0)))

    padpos = staged + iota
    plsc.store_scatter(stg_s, [padpos], iota)
    plsc.store_scatter(stg_d, [padpos], jnp.zeros((LANES,), jnp.int32))
    plsc.store_scatter(stg_w, [padpos], zv)
    staged = staged + ((LANES - (staged & (LANES - 1))) & (LANES - 1))

    def _pad_cond(cc):
        s, _ = cc
        return (s & (FL - 1)) != 0

    def _pad_body(cc):
        s, nb = cc
        pp = s + iota
        plsc.store_scatter(stg_s, [pp], iota)
        plsc.store_scatter(stg_d, [pp], jnp.zeros((LANES,), jnp.int32))
        plsc.store_scatter(stg_w, [pp], zv)
        return s + LANES, nb
    staged, nblk = lax.while_loop(_pad_cond, _pad_body, (staged, nblk))
    staged, nblk = _flush_when(staged, nblk, staged > 0)

    def _fold(dst_ref, q):
        s = pl.ds(q * LANES, LANES)

        def f(i, carry):
            dst_ref[i, s] = dst_ref[i, s] + rows[i, pl.ds(0, LANES)]
            return carry
        lax.fori_loop(0, BPT, f, 0)

    def _permute_ids(idref, qbase):
        def f(i, carry):
            s = pl.ds(i * LANES, LANES)
            v = idref[s]
            qid[s] = (v & (NS - 1)) * RPT + (v >> 4) + qbase
            return carry
        lax.fori_loop(0, BPT // LANES, f, 0)

    for q in range(QPC):
        qbase = (c * QPC + q) * NN
        _permute_ids(uidl, qbase)
        pltpu.sync_copy(x0.at[qid], rows.at[pl.ds(0, BPT)])
        _fold(bsu, q)
        _permute_ids(iidl, qbase)
        pltpu.sync_copy(x0.at[qid], rows.at[pl.ds(0, BPT)])
        _fold(bsi, q)

    for k in range(NL):
        srctab = x0 if k == 0 else xcur
        for q in range(QPC):
            qbase = (c * QPC + q) * NN

            def _zr(i, carry):
                acc[i, pl.ds(0, LANES)] = zv
                return carry
            lax.fori_loop(0, RPT, _zr, 0)

            def _chunk(j, carry):
                off = rbase + j * FL
                d1 = pltpu.async_copy(bsrc.at[pl.ds(off, FL)],
                                      sidx.at[pl.ds(0, FL)], sem1)
                d2 = pltpu.async_copy(bdst.at[pl.ds(off, FL)],
                                      didx.at[pl.ds(0, FL)], sem2)
                d3 = pltpu.async_copy(bw.at[pl.ds(off, FL)],
                                      wbuf.at[pl.ds(0, FL)], sem3)
                d1.wait()
                d2.wait()
                d3.wait()

                def _off(i, cc):
                    s = pl.ds(i * LANES, LANES)
                    v = sidx[s]
                    sidx[s] = (v & (NS - 1)) * RPT + (v >> 4) + qbase
                    return cc
                lax.fori_loop(0, FL // LANES, _off, 0)

                pltpu.sync_copy(srctab.at[sidx.at[pl.ds(0, FL)]], rows)

                def _edges(g, cc):
                    gb = g * LANES
                    evb = jnp.full((LANES,), gb, jnp.int32)
                    for e in range(LANES):
                        ev = evb + e
                        dl = plsc.load_gather(didx, [ev])
                        wb = plsc.load_gather(wbuf, [ev])
                        rv = rows[gb + e, pl.ds(0, LANES)]
                        plsc.addupdate_scatter(acc, [dl, iota], rv * wb)
                    return cc
                lax.fori_loop(0, FL // LANES, _edges, 0)
                return carry
            lax.fori_loop(0, nblk, _chunk, 0)
            plsc.subcore_barrier()

            pltpu.sync_copy(acc, xcur.at[pl.ds(qbase + t * RPT, RPT)])
            plsc.subcore_barrier()

            _permute_ids(uidl, qbase)
            pltpu.sync_copy(xcur.at[qid], rows.at[pl.ds(0, BPT)])
            _fold(bsu, q)
            _permute_ids(iidl, qbase)
            pltpu.sync_copy(xcur.at[qid], rows.at[pl.ds(0, BPT)])
            _fold(bsi, q)

    def _dot(g, carry):
        gb = g * LANES
        ridx = iota + gb
        accv = jnp.zeros((LANES,), jnp.float32)
        for dcol in range(DH):
            cidx = jnp.full((LANES,), dcol, jnp.int32)
            uv = plsc.load_gather(bsu, [ridx, cidx])
            iv = plsc.load_gather(bsi, [ridx, cidx])
            accv = accv + uv * iv
        ob[pl.ds(gb, LANES)] = accv * (1.0 / ((NL + 1) * (NL + 1)))
        return carry
    lax.fori_loop(0, BPT // LANES, _dot, 0)

    pltpu.sync_copy(ob, out.at[c, pl.ds(t * BPT, BPT)])


def kernel(users, items, user_emb, item_emb, edge_src, edge_dst, edge_w):
    all_emb = jnp.concatenate([user_emb, item_emb], axis=0)
    p = jnp.arange(NN, dtype=jnp.int32)
    inv = (p % RPT) * NS + p // RPT
    em = all_emb[inv]
    xq = em.reshape(NN, NQ, LANES).transpose(1, 0, 2).reshape(NQ * NN, LANES)
    part = _ltocf_sc(users, items, xq, edge_src, edge_dst, edge_w)
    return part[0] + part[1]


# speedup vs baseline: 1.7006x; 1.1749x over previous
import functools

import jax
import jax.numpy as jnp
from jax import lax
from jax.experimental import pallas as pl
from jax.experimental.pallas import tpu as pltpu
from jax.experimental.pallas import tpu_sc as plsc

N_USERS = 15000
N_ITEMS = 35000
NN = N_USERS + N_ITEMS
E = 800000
D = 64
NL = 4
B = 4096

NC = 2
NS = 16
LANES = 16
NQ = D // LANES
QPC = NQ // NC
DH = D // NC
RPT = NN // NS
BPT = B // NS

SCH = 4000
NSCH = E // SCH
FL = 2048
BCAP = 392 * FL
MAXBLK = BCAP // FL

_mesh = plsc.VectorSubcoreMesh(core_axis_name="c", subcore_axis_name="s")


@functools.partial(
    pl.kernel,
    out_type=jax.ShapeDtypeStruct((NC, B), jnp.float32),
    mesh=_mesh,
    compiler_params=pltpu.CompilerParams(needs_layout_passes=False,
                                         use_tc_tiling_on_sc=False),
    scratch_types=[
        pltpu.HBM((NQ * NN, LANES), jnp.float32),
        pltpu.HBM((NC * NS * BCAP,), jnp.int32),
        pltpu.HBM((NC * NS * BCAP,), jnp.int32),
        pltpu.HBM((NC * NS * BCAP,), jnp.float32),
        pltpu.VMEM((RPT, LANES), jnp.float32),
        pltpu.VMEM((FL, LANES), jnp.float32),
        pltpu.VMEM((SCH,), jnp.int32),
        pltpu.VMEM((SCH,), jnp.int32),
        pltpu.VMEM((SCH,), jnp.float32),
        pltpu.VMEM((FL + LANES,), jnp.int32),
        pltpu.VMEM((FL + LANES,), jnp.int32),
        pltpu.VMEM((FL + LANES,), jnp.float32),
        pltpu.VMEM((BPT, DH), jnp.float32),
        pltpu.VMEM((BPT, DH), jnp.float32),
        pltpu.VMEM((BPT,), jnp.int32),
        pltpu.VMEM((BPT,), jnp.int32),
        pltpu.VMEM((BPT,), jnp.int32),
        pltpu.VMEM((BPT,), jnp.float32),
        pltpu.SemaphoreType.DMA,
        pltpu.SemaphoreType.DMA,
        pltpu.SemaphoreType.DMA,
    ],
)
def _ltocf_sc(users, items, x0, src, dst, w, out,
              xcur, bsrc, bdst, bw, acc, rows, sidx, didx, wbuf,
              stg_s, stg_d, stg_w, bsu, bsi, uidl, iidl, qid, ob,
              sem1, sem2, sem3):
    c = lax.axis_index("c")
    t = lax.axis_index("s")
    iota = lax.iota(jnp.int32, LANES)
    zv = jnp.zeros((LANES,), jnp.float32)
    rbase = (c * NS + t) * BCAP

    pltpu.sync_copy(users.at[pl.ds(t * BPT, BPT)], uidl)
    pltpu.sync_copy(items.at[pl.ds(t * BPT, BPT)], iidl)

    def _prep(i, carry):
        s = pl.ds(i * LANES, LANES)
        iidl[s] = iidl[s] + N_USERS
        return carry
    lax.fori_loop(0, BPT // LANES, _prep, 0)

    def _zb(i, carry):
        bsu[i, pl.ds(0, LANES)] = zv
        bsu[i, pl.ds(LANES, LANES)] = zv
        bsi[i, pl.ds(0, LANES)] = zv
        bsi[i, pl.ds(LANES, LANES)] = zv
        return carry
    lax.fori_loop(0, BPT, _zb, 0)

    def _scan_chunk(js, carry):
        ebase = js * SCH
        d1 = pltpu.async_copy(src.at[pl.ds(ebase, SCH)], sidx, sem1)
        d2 = pltpu.async_copy(dst.at[pl.ds(ebase, SCH)], didx, sem2)
        d3 = pltpu.async_copy(w.at[pl.ds(ebase, SCH)], wbuf, sem3)
        d1.wait()
        d2.wait()
        d3.wait()

        def _vreg(g, cc):
            staged, nblk = cc
            s = pl.ds(g * LANES, LANES)
            sv = sidx[s]
            dv = didx[s]
            wv = wbuf[s]
            m = (dv & (NS - 1)) == t
            mi = m.astype(jnp.int32)
            rank = plsc.cumsum(mi)
            cnt = rank[15]
            pos = rank + (staged - 1)
            plsc.store_scatter(stg_s, [pos], sv, mask=m)
            plsc.store_scatter(stg_d, [pos], dv >> 4, mask=m)
            plsc.store_scatter(stg_w, [pos], wv, mask=m)
            staged = staged + cnt

            full = staged >= FL
            fs, fn = _flush_when(staged, nblk, full)
            return fs, fn
        return lax.fori_loop(0, SCH // LANES, _vreg, carry)

    def _flush_when(staged, nblk, cond):
        @pl.when(cond)
        def _():
            off = rbase + nblk * FL
            pltpu.sync_copy(stg_s.at[pl.ds(0, FL)], bsrc.at[pl.ds(off, FL)])
            pltpu.sync_copy(stg_d.at[pl.ds(0, FL)], bdst.at[pl.ds(off, FL)])
            pltpu.sync_copy(stg_w.at[pl.ds(0, FL)], bw.at[pl.ds(off, FL)])
            stg_s[pl.ds(0, LANES)] = stg_s[pl.ds(FL, LANES)]
            stg_d[pl.ds(0, LANES)] = stg_d[pl.ds(FL, LANES)]
            stg_w[pl.ds(0, LANES)] = stg_w[pl.ds(FL, LANES)]
        return (jnp.where(cond, staged - FL, staged),
                jnp.where(cond, nblk + 1, nblk))

    staged, nblk = lax.fori_loop(0, NSCH, _scan_chunk,
                                 (jnp.int32(0), jnp.int32(0)))

    padpos = staged + iota
    plsc.store_scatter(stg_s, [padpos], iota)
    plsc.store_scatter(stg_d, [padpos], jnp.zeros((LANES,), jnp.int32))
    plsc.store_scatter(stg_w, [padpos], zv)
    staged = staged + ((LANES - (staged & (LANES - 1))) & (LANES - 1))

    def _pad_cond(cc):
        s, _ = cc
        return (s & (FL - 1)) != 0

    def _pad_body(cc):
        s, nb = cc
        pp = s + iota
        plsc.store_scatter(stg_s, [pp], iota)
        plsc.store_scatter(stg_d, [pp], jnp.zeros((LANES,), jnp.int32))
        plsc.store_scatter(stg_w, [pp], zv)
        return s + LANES, nb
    staged, nblk = lax.while_loop(_pad_cond, _pad_body, (staged, nblk))
    staged, nblk = _flush_when(staged, nblk, staged > 0)

    def _fold(dst_ref, q):
        s = pl.ds(q * LANES, LANES)

        def f(i, carry):
            dst_ref[i, s] = dst_ref[i, s] + rows[i, pl.ds(0, LANES)]
            return carry
        lax.fori_loop(0, BPT, f, 0)

    def _permute_ids(idref, qbase):
        def f(i, carry):
            s = pl.ds(i * LANES, LANES)
            v = idref[s]
            qid[s] = (v & (NS - 1)) * RPT + (v >> 4) + qbase
            return carry
        lax.fori_loop(0, BPT // LANES, f, 0)

    for q in range(QPC):
        qbase = (c * QPC + q) * NN
        _permute_ids(uidl, qbase)
        pltpu.sync_copy(x0.at[qid], rows.at[pl.ds(0, BPT)])
        _fold(bsu, q)
        _permute_ids(iidl, qbase)
        pltpu.sync_copy(x0.at[qid], rows.at[pl.ds(0, BPT)])
        _fold(bsi, q)

    for k in range(NL):
        srctab = x0 if k == 0 else xcur
        for q in range(QPC):
            qbase = (c * QPC + q) * NN

            def _zr(i, carry):
                acc[i, pl.ds(0, LANES)] = zv
                return carry
            lax.fori_loop(0, RPT, _zr, 0)

            def _chunk(j, carry):
                off = rbase + j * FL
                d1 = pltpu.async_copy(bsrc.at[pl.ds(off, FL)],
                                      sidx.at[pl.ds(0, FL)], sem1)
                d2 = pltpu.async_copy(bdst.at[pl.ds(off, FL)],
                                      didx.at[pl.ds(0, FL)], sem2)
                d3 = pltpu.async_copy(bw.at[pl.ds(off, FL)],
                                      wbuf.at[pl.ds(0, FL)], sem3)
                d1.wait()
                d2.wait()
                d3.wait()

                def _off(i, cc):
                    s = pl.ds(i * LANES, LANES)
                    v = sidx[s]
                    sidx[s] = (v & (NS - 1)) * RPT + (v >> 4) + qbase
                    return cc
                lax.fori_loop(0, FL // LANES, _off, 0)

                pltpu.sync_copy(srctab.at[sidx.at[pl.ds(0, FL)]], rows)

                def _edges(g, cc):
                    gb = g * LANES
                    s = pl.ds(gb, LANES)
                    dl16 = didx[s]
                    w16 = wbuf[s]
                    for e in range(LANES):
                        dl = jnp.full((LANES,), dl16[e], jnp.int32)
                        wb = jnp.full((LANES,), w16[e], jnp.float32)
                        rv = rows[gb + e, pl.ds(0, LANES)]
                        plsc.addupdate_scatter(acc, [dl, iota], rv * wb)
                    return cc
                lax.fori_loop(0, FL // LANES, _edges, 0)
                return carry
            lax.fori_loop(0, nblk, _chunk, 0)
            plsc.subcore_barrier()

            pltpu.sync_copy(acc, xcur.at[pl.ds(qbase + t * RPT, RPT)])
            plsc.subcore_barrier()

            _permute_ids(uidl, qbase)
            pltpu.sync_copy(xcur.at[qid], rows.at[pl.ds(0, BPT)])
            _fold(bsu, q)
            _permute_ids(iidl, qbase)
            pltpu.sync_copy(xcur.at[qid], rows.at[pl.ds(0, BPT)])
            _fold(bsi, q)

    def _dot(g, carry):
        gb = g * LANES
        ridx = iota + gb
        accv = jnp.zeros((LANES,), jnp.float32)
        for dcol in range(DH):
            cidx = jnp.full((LANES,), dcol, jnp.int32)
            uv = plsc.load_gather(bsu, [ridx, cidx])
            iv = plsc.load_gather(bsi, [ridx, cidx])
            accv = accv + uv * iv
        ob[pl.ds(gb, LANES)] = accv * (1.0 / ((NL + 1) * (NL + 1)))
        return carry
    lax.fori_loop(0, BPT // LANES, _dot, 0)

    pltpu.sync_copy(ob, out.at[c, pl.ds(t * BPT, BPT)])


def kernel(users, items, user_emb, item_emb, edge_src, edge_dst, edge_w):
    all_emb = jnp.concatenate([user_emb, item_emb], axis=0)
    p = jnp.arange(NN, dtype=jnp.int32)
    inv = (p % RPT) * NS + p // RPT
    em = all_emb[inv]
    xq = em.reshape(NN, NQ, LANES).transpose(1, 0, 2).reshape(NQ * NN, LANES)
    part = _ltocf_sc(users, items, xq, edge_src, edge_dst, edge_w)
    return part[0] + part[1]


# speedup vs baseline: 2.0185x; 1.1869x over previous
import functools

import jax
import jax.numpy as jnp
from jax import lax
from jax.experimental import pallas as pl
from jax.experimental.pallas import tpu as pltpu
from jax.experimental.pallas import tpu_sc as plsc

N_USERS = 15000
N_ITEMS = 35000
NN = N_USERS + N_ITEMS
E = 800000
D = 64
NL = 4
B = 4096

NC = 2
NS = 16
LANES = 16
NQ = D // LANES
QPC = NQ // NC
DH = D // NC
RPT = NN // NS
BPT = B // NS

SCH = 3200
NSCH = E // SCH
FL = 2048
BCAP = 392 * FL
MAXBLK = BCAP // FL

_mesh = plsc.VectorSubcoreMesh(core_axis_name="c", subcore_axis_name="s")


@functools.partial(
    pl.kernel,
    out_type=jax.ShapeDtypeStruct((NC, B), jnp.float32),
    mesh=_mesh,
    compiler_params=pltpu.CompilerParams(needs_layout_passes=False,
                                         use_tc_tiling_on_sc=False),
    scratch_types=[
        pltpu.HBM((NQ * NN, LANES), jnp.float32),
        pltpu.HBM((NC * NS * BCAP,), jnp.int32),
        pltpu.HBM((NC * NS * BCAP,), jnp.int32),
        pltpu.HBM((NC * NS * BCAP,), jnp.float32),
        pltpu.VMEM((RPT, LANES), jnp.float32),
        pltpu.VMEM((FL, LANES), jnp.float32),
        pltpu.VMEM((SCH,), jnp.int32),
        pltpu.VMEM((SCH,), jnp.int32),
        pltpu.VMEM((SCH,), jnp.float32),
        pltpu.VMEM((SCH,), jnp.int32),
        pltpu.VMEM((SCH,), jnp.int32),
        pltpu.VMEM((SCH,), jnp.float32),
        pltpu.VMEM((FL + LANES,), jnp.int32),
        pltpu.VMEM((FL + LANES,), jnp.int32),
        pltpu.VMEM((FL + LANES,), jnp.float32),
        pltpu.VMEM((BPT, DH), jnp.float32),
        pltpu.VMEM((BPT, DH), jnp.float32),
        pltpu.VMEM((BPT,), jnp.int32),
        pltpu.VMEM((BPT,), jnp.int32),
        pltpu.VMEM((BPT,), jnp.int32),
        pltpu.VMEM((BPT,), jnp.float32),
        pltpu.SemaphoreType.DMA,
        pltpu.SemaphoreType.DMA,
        pltpu.SemaphoreType.DMA,
        pltpu.SemaphoreType.DMA,
        pltpu.SemaphoreType.DMA,
        pltpu.SemaphoreType.DMA,
    ],
)
def _ltocf_sc(users, items, x0, src, dst, w, out,
              xcur, bsrc, bdst, bw, acc, rows, sidx, didx, wbuf,
              sidx2, didx2, wbuf2,
              stg_s, stg_d, stg_w, bsu, bsi, uidl, iidl, qid, ob,
              sem1, sem2, sem3, sem4, sem5, sem6):
    c = lax.axis_index("c")
    t = lax.axis_index("s")
    iota = lax.iota(jnp.int32, LANES)
    zv = jnp.zeros((LANES,), jnp.float32)
    rbase = (c * NS + t) * BCAP

    pltpu.sync_copy(users.at[pl.ds(t * BPT, BPT)], uidl)
    pltpu.sync_copy(items.at[pl.ds(t * BPT, BPT)], iidl)

    def _prep(i, carry):
        s = pl.ds(i * LANES, LANES)
        iidl[s] = iidl[s] + N_USERS
        return carry
    lax.fori_loop(0, BPT // LANES, _prep, 0)

    def _zb(i, carry):
        bsu[i, pl.ds(0, LANES)] = zv
        bsu[i, pl.ds(LANES, LANES)] = zv
        bsi[i, pl.ds(0, LANES)] = zv
        bsi[i, pl.ds(LANES, LANES)] = zv
        return carry
    lax.fori_loop(0, BPT, _zb, 0)

    _bufsets = ((sidx, didx, wbuf, sem1, sem2, sem3),
                (sidx2, didx2, wbuf2, sem4, sem5, sem6))

    def _scan_start(j, bs):
        ebase = j * SCH
        pltpu.async_copy(src.at[pl.ds(ebase, SCH)], bs[0], bs[3])
        pltpu.async_copy(dst.at[pl.ds(ebase, SCH)], bs[1], bs[4])
        pltpu.async_copy(w.at[pl.ds(ebase, SCH)], bs[2], bs[5])

    def _scan_wait(bs):
        pltpu.make_async_copy(src.at[pl.ds(0, SCH)], bs[0], bs[3]).wait()
        pltpu.make_async_copy(dst.at[pl.ds(0, SCH)], bs[1], bs[4]).wait()
        pltpu.make_async_copy(w.at[pl.ds(0, SCH)], bs[2], bs[5]).wait()

    def _scan_body(sb, db, wb, carry):
        def _vreg2(g2, cc):
            staged, nblk = cc
            s1 = pl.ds(g2 * 2 * LANES, LANES)
            s2 = pl.ds(g2 * 2 * LANES + LANES, LANES)
            sv1, dv1, wv1 = sb[s1], db[s1], wb[s1]
            sv2, dv2, wv2 = sb[s2], db[s2], wb[s2]
            m1 = (dv1 & (NS - 1)) == t
            m2 = (dv2 & (NS - 1)) == t
            r1 = plsc.cumsum(m1.astype(jnp.int32))
            r2 = plsc.cumsum(m2.astype(jnp.int32))

            pos1 = r1 + (staged - 1)
            plsc.store_scatter(stg_s, [pos1], sv1, mask=m1)
            plsc.store_scatter(stg_d, [pos1], dv1 >> 4, mask=m1)
            plsc.store_scatter(stg_w, [pos1], wv1, mask=m1)
            staged = staged + r1[15]
            staged, nblk = _flush_when(staged, nblk, staged >= FL)

            pos2 = r2 + (staged - 1)
            plsc.store_scatter(stg_s, [pos2], sv2, mask=m2)
            plsc.store_scatter(stg_d, [pos2], dv2 >> 4, mask=m2)
            plsc.store_scatter(stg_w, [pos2], wv2, mask=m2)
            staged = staged + r2[15]
            return _flush_when(staged, nblk, staged >= FL)
        return lax.fori_loop(0, SCH // (2 * LANES), _vreg2, carry)

    def _flush_when(staged, nblk, cond):
        @pl.when(cond)
        def _():
            off = rbase + nblk * FL
            pltpu.sync_copy(stg_s.at[pl.ds(0, FL)], bsrc.at[pl.ds(off, FL)])
            pltpu.sync_copy(stg_d.at[pl.ds(0, FL)], bdst.at[pl.ds(off, FL)])
            pltpu.sync_copy(stg_w.at[pl.ds(0, FL)], bw.at[pl.ds(off, FL)])
            stg_s[pl.ds(0, LANES)] = stg_s[pl.ds(FL, LANES)]
            stg_d[pl.ds(0, LANES)] = stg_d[pl.ds(FL, LANES)]
            stg_w[pl.ds(0, LANES)] = stg_w[pl.ds(FL, LANES)]
        return (jnp.where(cond, staged - FL, staged),
                jnp.where(cond, nblk + 1, nblk))

    _scan_start(0, _bufsets[0])
    _scan_start(1, _bufsets[1])

    def _scan_outer(jj, carry):
        for b in range(2):
            bs = _bufsets[b]
            _scan_wait(bs)
            carry = _scan_body(bs[0], bs[1], bs[2], carry)
            j = jj * 2 + b

            @pl.when(j + 2 < NSCH)
            def _():
                _scan_start(j + 2, bs)
        return carry
    staged, nblk = lax.fori_loop(0, NSCH // 2, _scan_outer,
                                 (jnp.int32(0), jnp.int32(0)))

    padpos = staged + iota
    plsc.store_scatter(stg_s, [padpos], iota)
    plsc.store_scatter(stg_d, [padpos], jnp.zeros((LANES,), jnp.int32))
    plsc.store_scatter(stg_w, [padpos], zv)
    staged = staged + ((LANES - (staged & (LANES - 1))) & (LANES - 1))

    def _pad_cond(cc):
        s, _ = cc
        return (s & (FL - 1)) != 0

    def _pad_body(cc):
        s, nb = cc
        pp = s + iota
        plsc.store_scatter(stg_s, [pp], iota)
        plsc.store_scatter(stg_d, [pp], jnp.zeros((LANES,), jnp.int32))
        plsc.store_scatter(stg_w, [pp], zv)
        return s + LANES, nb
    staged, nblk = lax.while_loop(_pad_cond, _pad_body, (staged, nblk))
    staged, nblk = _flush_when(staged, nblk, staged > 0)

    def _fold(dst_ref, q):
        s = pl.ds(q * LANES, LANES)

        def f(i, carry):
            dst_ref[i, s] = dst_ref[i, s] + rows[i, pl.ds(0, LANES)]
            return carry
        lax.fori_loop(0, BPT, f, 0)

    def _permute_ids(idref, qbase):
        def f(i, carry):
            s = pl.ds(i * LANES, LANES)
            v = idref[s]
            qid[s] = (v & (NS - 1)) * RPT + (v >> 4) + qbase
            return carry
        lax.fori_loop(0, BPT // LANES, f, 0)

    for q in range(QPC):
        qbase = (c * QPC + q) * NN
        _permute_ids(uidl, qbase)
        pltpu.sync_copy(x0.at[qid], rows.at[pl.ds(0, BPT)])
        _fold(bsu, q)
        _permute_ids(iidl, qbase)
        pltpu.sync_copy(x0.at[qid], rows.at[pl.ds(0, BPT)])
        _fold(bsi, q)

    for k in range(NL):
        srctab = x0 if k == 0 else xcur
        for q in range(QPC):
            qbase = (c * QPC + q) * NN

            def _zr(i, carry):
                acc[i, pl.ds(0, LANES)] = zv
                return carry
            lax.fori_loop(0, RPT, _zr, 0)

            def _pass_start(j, bs):
                off = rbase + j * FL
                pltpu.async_copy(bsrc.at[pl.ds(off, FL)],
                                 bs[0].at[pl.ds(0, FL)], bs[3])
                pltpu.async_copy(bdst.at[pl.ds(off, FL)],
                                 bs[1].at[pl.ds(0, FL)], bs[4])
                pltpu.async_copy(bw.at[pl.ds(off, FL)],
                                 bs[2].at[pl.ds(0, FL)], bs[5])

            def _pass_wait(bs):
                pltpu.make_async_copy(bsrc.at[pl.ds(0, FL)],
                                     bs[0].at[pl.ds(0, FL)], bs[3]).wait()
                pltpu.make_async_copy(bdst.at[pl.ds(0, FL)],
                                     bs[1].at[pl.ds(0, FL)], bs[4]).wait()
                pltpu.make_async_copy(bw.at[pl.ds(0, FL)],
                                     bs[2].at[pl.ds(0, FL)], bs[5]).wait()

            @pl.when(nblk > 0)
            def _():
                _pass_start(0, _bufsets[0])

            @pl.when(nblk > 1)
            def _():
                _pass_start(1, _bufsets[1])

            def _chunk2(jj, carry):
                for b in range(2):
                    bs = _bufsets[b]
                    j = jj * 2 + b

                    @pl.when(j < nblk)
                    def _(bs=bs, j=j):
                        _pass_wait(bs)
                        sb, db_, wb_ = bs[0], bs[1], bs[2]

                        def _off(i, cc):
                            s = pl.ds(i * LANES, LANES)
                            v = sb[s]
                            sb[s] = (v & (NS - 1)) * RPT + (v >> 4) + qbase
                            return cc
                        lax.fori_loop(0, FL // LANES, _off, 0)

                        pltpu.sync_copy(srctab.at[sb.at[pl.ds(0, FL)]], rows)

                        def _edges(g, cc):
                            gb = g * LANES
                            s = pl.ds(gb, LANES)
                            dl16 = db_[s]
                            w16 = wb_[s]
                            for e in range(LANES):
                                dl = jnp.full((LANES,), dl16[e], jnp.int32)
                                wvv = jnp.full((LANES,), w16[e], jnp.float32)
                                rv = rows[gb + e, pl.ds(0, LANES)]
                                plsc.addupdate_scatter(acc, [dl, iota],
                                                       rv * wvv)
                            return cc
                        lax.fori_loop(0, FL // LANES, _edges, 0)

                        @pl.when(j + 2 < nblk)
                        def _():
                            _pass_start(j + 2, bs)
                return carry
            lax.fori_loop(0, (nblk + 1) // 2, _chunk2, 0)
            plsc.subcore_barrier()

            pltpu.sync_copy(acc, xcur.at[pl.ds(qbase + t * RPT, RPT)])
            plsc.subcore_barrier()

            _permute_ids(uidl, qbase)
            pltpu.sync_copy(xcur.at[qid], rows.at[pl.ds(0, BPT)])
            _fold(bsu, q)
            _permute_ids(iidl, qbase)
            pltpu.sync_copy(xcur.at[qid], rows.at[pl.ds(0, BPT)])
            _fold(bsi, q)

    def _dot(g, carry):
        gb = g * LANES
        ridx = iota + gb
        accv = jnp.zeros((LANES,), jnp.float32)
        for dcol in range(DH):
            cidx = jnp.full((LANES,), dcol, jnp.int32)
            uv = plsc.load_gather(bsu, [ridx, cidx])
            iv = plsc.load_gather(bsi, [ridx, cidx])
            accv = accv + uv * iv
        ob[pl.ds(gb, LANES)] = accv * (1.0 / ((NL + 1) * (NL + 1)))
        return carry
    lax.fori_loop(0, BPT // LANES, _dot, 0)

    pltpu.sync_copy(ob, out.at[c, pl.ds(t * BPT, BPT)])


def kernel(users, items, user_emb, item_emb, edge_src, edge_dst, edge_w):
    all_emb = jnp.concatenate([user_emb, item_emb], axis=0)
    p = jnp.arange(NN, dtype=jnp.int32)
    inv = (p % RPT) * NS + p // RPT
    em = all_emb[inv]
    xq = em.reshape(NN, NQ, LANES).transpose(1, 0, 2).reshape(NQ * NN, LANES)
    part = _ltocf_sc(users, items, xq, edge_src, edge_dst, edge_w)
    return part[0] + part[1]
